# Initial kernel scaffold; baseline (speedup 1.0000x reference)
import jax
import jax.numpy as jnp
from jax.experimental import pallas as pl


def kernel(x, modality_ids, weight):
    raise NotImplementedError("write your pallas kernel here")


# speedup vs baseline: 2.2832x; 2.2832x over previous
import functools

import jax
import jax.numpy as jnp
from jax.experimental import pallas as pl
from jax.experimental.pallas import tpu as pltpu

NUM_EXPERTS = 3
IN_FEATURES = 1024
OUT_FEATURES = 1024
N_TOKENS = 8192
TOKEN_BLOCK = 1024


def _body(x_ref, ids_ref, w_ref, out_ref):
    x = x_ref[...]
    ids = ids_ref[...]
    acc = jnp.zeros((x.shape[0], OUT_FEATURES), jnp.float32)
    for e in range(NUM_EXPERTS):
        y = jax.lax.dot_general(
            x, w_ref[e],
            dimension_numbers=(((1,), (1,)), ((), ())),
            preferred_element_type=jnp.float32,
        )
        acc = jnp.where(ids == float(e), y, acc)
    out_ref[...] = acc


def kernel(x, modality_ids, weight):
    w = weight.reshape(NUM_EXPERTS, OUT_FEATURES, IN_FEATURES)
    ids_f = modality_ids.astype(jnp.float32).reshape(N_TOKENS, 1)
    nb = N_TOKENS // TOKEN_BLOCK
    return pl.pallas_call(
        _body,
        grid=(nb,),
        in_specs=[
            pl.BlockSpec((TOKEN_BLOCK, IN_FEATURES), lambda i: (i, 0)),
            pl.BlockSpec((TOKEN_BLOCK, 1), lambda i: (i, 0)),
            pl.BlockSpec(
                (NUM_EXPERTS, OUT_FEATURES, IN_FEATURES), lambda i: (0, 0, 0)
            ),
        ],
        out_specs=pl.BlockSpec((TOKEN_BLOCK, OUT_FEATURES), lambda i: (i, 0)),
        out_shape=jax.ShapeDtypeStruct((N_TOKENS, OUT_FEATURES), jnp.float32),
    )(x, ids_f, w)
